# Initial kernel scaffold; baseline (speedup 1.0000x reference)
import jax
import jax.numpy as jnp
from jax.experimental import pallas as pl


def kernel(feat, image_shapes, conv_w, conv_b, cls_w, cls_b, reg_w, reg_b):
    raise NotImplementedError("write your pallas kernel here")


# speedup vs baseline: 28.3339x; 28.3339x over previous
import math

import jax
import jax.numpy as jnp
import numpy as np
from jax.experimental import pallas as pl
from jax.experimental.pallas import tpu as pltpu

IN_CHANNELS = 256
NUM_ANCHORS = 9
PRE_NMS_TOPK = 2000
POST_NMS_TOPK = 1000
NMS_THRESH = 0.7
SCALES = (128.0, 256.0, 512.0)
RATIOS = (0.5, 1.0, 2.0)
STRIDE = 16.0
BBOX_CLIP = math.log(1000.0 / 16.0)

NSEL = 2048
NOUT = 1024


def _np_anchor_planes(H, W):
    scales = np.array(SCALES, dtype=np.float32)
    ratios = np.array(RATIOS, dtype=np.float32)
    h_r = np.sqrt(ratios)
    w_r = (np.float32(1.0) / h_r).astype(np.float32)
    ws = (w_r[:, None] * scales[None, :]).reshape(-1)
    hs = (h_r[:, None] * scales[None, :]).reshape(-1)
    base = np.stack([-ws / np.float32(2.0), -hs / np.float32(2.0),
                     ws / np.float32(2.0), hs / np.float32(2.0)], axis=1)
    sx = ((np.arange(W, dtype=np.float32) + np.float32(0.5)) * np.float32(STRIDE))
    sy = ((np.arange(H, dtype=np.float32) + np.float32(0.5)) * np.float32(STRIDE))
    yy, xx = np.meshgrid(sy, sx, indexing='ij')
    xf, yf = xx.reshape(-1), yy.reshape(-1)
    x1 = (xf[None, :] + base[:, 0:1]).astype(np.float32)
    y1 = (yf[None, :] + base[:, 1:2]).astype(np.float32)
    x2 = (xf[None, :] + base[:, 2:3]).astype(np.float32)
    y2 = (yf[None, :] + base[:, 3:4]).astype(np.float32)
    return x1, y1, x2, y2


def _np_anchors_out(H, W):
    x1, y1, x2, y2 = _np_anchor_planes(H, W)
    a = np.stack([x1, y1, x2, y2], axis=2)
    return np.transpose(a, (1, 0, 2)).reshape(-1, 4)


def _rowb(row, R):
    return jax.lax.dot_general(
        jnp.ones((R, 1), jnp.float32), row, (((1,), (0,)), ((), ())),
        preferred_element_type=jnp.float32,
        precision=jax.lax.Precision.HIGHEST)


def _trunk_body(x_ref, w_ref, b_ref, o_ref):
    y = jax.lax.dot_general(x_ref[...], w_ref[...], (((1,), (0,)), ((), ())),
                            preferred_element_type=jnp.float32)
    o_ref[...] = jnp.maximum(y + _rowb(b_ref[...], y.shape[0]), 0.0)


def _heads_body(r_ref, cw_ref, cb_ref, rw_ref, rb_ref, cls_ref, reg_ref):
    rpn = jnp.transpose(r_ref[...])
    cls_ref[...] = jax.lax.dot_general(
        cw_ref[...], rpn, (((1,), (0,)), ((), ())),
        preferred_element_type=jnp.float32) + cb_ref[...]
    reg_ref[...] = jax.lax.dot_general(
        rw_ref[...], rpn, (((1,), (0,)), ((), ())),
        preferred_element_type=jnp.float32) + rb_ref[...]


def _excl_cumsum_lanes(x):
    n = x.shape[1]
    incl = x
    s = 1
    while s < n:
        shifted = jnp.concatenate(
            [jnp.zeros((1, s), incl.dtype), incl[:, :n - s]], axis=1)
        incl = incl + shifted
        s *= 2
    return incl - x


def _orderable(msc):
    b = jax.lax.bitcast_convert_type(msc, jnp.int32)
    return jnp.where(b >= 0, b, b ^ jnp.int32(0x7FFFFFFF))


def _select_body(cls_ref, reg_ref, anc_ref, img_ref, out_ref, sup_ref):
    NA = cls_ref.shape[1]
    NTOT = anc_ref.shape[1]

    def ext(row):
        return jnp.concatenate(
            [row, jnp.zeros((1, NTOT - NA), jnp.float32)], axis=1)

    logits = ext(cls_ref[...])
    reg = reg_ref[...]
    dx = ext(reg[0:1, :])
    dy = ext(reg[1:2, :])
    dw = ext(reg[2:3, :])
    dh = ext(reg[3:4, :])
    ax1 = anc_ref[0:1, :]
    ay1 = anc_ref[1:2, :]
    ax2 = anc_ref[2:3, :]
    ay2 = anc_ref[3:4, :]
    img = img_ref[...]
    img_h = img[0:1, 0:1]
    img_w = img[0:1, 1:2]
    clip = jnp.float32(BBOX_CLIP)
    w = ax2 - ax1
    h = ay2 - ay1
    cx = ax1 + 0.5 * w
    cy = ay1 + 0.5 * h
    dwc = jnp.minimum(dw, clip)
    dhc = jnp.minimum(dh, clip)
    pcx = dx * w + cx
    pcy = dy * h + cy
    pw = jnp.exp(dwc) * w
    ph = jnp.exp(dhc) * h
    x1 = pcx - 0.5 * pw
    y1 = pcy - 0.5 * ph
    x2 = pcx + 0.5 * pw
    y2 = pcy + 0.5 * ph
    x1 = jnp.minimum(jnp.maximum(x1, 0.0), img_w)
    y1 = jnp.minimum(jnp.maximum(y1, 0.0), img_h)
    x2 = jnp.minimum(jnp.maximum(x2, 0.0), img_w)
    y2 = jnp.minimum(jnp.maximum(y2, 0.0), img_h)
    sc = 1.0 / (1.0 + jnp.exp(-logits))
    validb = ((x2 - x1) >= 1e-2) & ((y2 - y1) >= 1e-2)
    msc = jnp.where(validb, sc, -1.0)
    lane = jax.lax.broadcasted_iota(jnp.int32, (1, NTOT), 1)
    msc = jnp.where(lane < NA, msc, -2.0)
    idxf = lane.astype(jnp.float32)
    x8 = jnp.concatenate(
        [x1, y1, x2, y2, msc, idxf, jnp.zeros((2, NTOT), jnp.float32)], axis=0)
    u = _orderable(msc)

    def bit_step(i, T):
        b = 31 - i
        cand = T | (jnp.int32(1) << b)
        tcmp = cand ^ jnp.int32(-2147483648)
        cnt = jnp.sum((u >= tcmp).astype(jnp.int32))
        return jnp.where(cnt >= PRE_NMS_TOPK, cand, T)

    T = jax.lax.fori_loop(0, 32, bit_step, jnp.int32(0))
    t_s = T ^ jnp.int32(-2147483648)
    gt = (u > t_s)
    eq = (u == t_s)
    count_gt = jnp.sum(gt.astype(jnp.float32))
    m = PRE_NMS_TOPK - count_gt
    tiecnt = _excl_cumsum_lanes(eq.astype(jnp.float32))
    sel = gt | (eq & (tiecnt < m))
    self32 = sel.astype(jnp.float32)

    pos = _excl_cumsum_lanes(self32)
    JT = 1024
    qio2 = jax.lax.broadcasted_iota(jnp.int32, (JT, NSEL), 1).astype(jnp.float32)
    acc = jnp.zeros((8, NSEL), jnp.float32)
    for jt in range(NTOT // JT):
        sl = slice(jt * JT, (jt + 1) * JT)
        pos_c = jnp.transpose(pos[:, sl])
        sel_c = jnp.transpose(self32[:, sl])
        oh = ((pos_c == qio2) & (sel_c > 0.0)).astype(jnp.float32)
        acc = acc + jax.lax.dot_general(
            x8[:, sl], oh, (((1,), (0,)), ((), ())),
            preferred_element_type=jnp.float32,
            precision=jax.lax.Precision.HIGHEST)
    qrow = jax.lax.broadcasted_iota(jnp.int32, (1, NSEL), 1).astype(jnp.float32)
    padq = qrow >= float(PRE_NMS_TOPK)
    cx1 = acc[0:1, :]
    cy1 = acc[1:2, :]
    cx2 = acc[2:3, :]
    cy2 = acc[3:4, :]
    cmsc = jnp.where(padq, -3.0, acc[4:5, :])
    cidx = jnp.where(padq, 50000.0 + qrow, acc[5:6, :])

    RT = 256
    mscb = _rowb(cmsc, RT)
    idxb = _rowb(cidx, RT)
    rparts = []
    for rt in range(NSEL // RT):
        sl = slice(rt * RT, (rt + 1) * RT)
        msc_col = jnp.transpose(cmsc[:, sl])
        i_col = jnp.transpose(cidx[:, sl])
        before = (mscb > msc_col) | ((mscb == msc_col) & (idxb < i_col))
        r_col = jnp.sum(before.astype(jnp.float32), axis=1, keepdims=True)
        rparts.append(jnp.transpose(r_col))
    rank = jnp.concatenate(rparts, axis=1)

    vals = jnp.concatenate([cx1, cy1, cx2, cy2, cmsc, cidx,
                            jnp.zeros((2, NSEL), jnp.float32)], axis=0)
    valsT = jnp.transpose(vals)
    rankb = _rowb(rank, RT)
    sparts = []
    for rt in range(NSEL // RT):
        r0 = float(rt * RT)
        rio = jax.lax.broadcasted_iota(jnp.int32, (RT, 1), 0).astype(jnp.float32) + r0
        oh = (rio == rankb).astype(jnp.float32)
        sparts.append(jax.lax.dot_general(
            oh, valsT, (((1,), (0,)), ((), ())),
            preferred_element_type=jnp.float32,
            precision=jax.lax.Precision.HIGHEST))
    sortedT = jnp.concatenate(sparts, axis=0)
    srows = jnp.transpose(sortedT)
    sx1 = srows[0:1, :]
    sy1 = srows[1:2, :]
    sx2 = srows[2:3, :]
    sy2 = srows[3:4, :]
    ssc = srows[4:5, :]

    area_r = (sx2 - sx1) * (sy2 - sy1)
    jio = jax.lax.broadcasted_iota(jnp.int32, (1, NSEL), 1).astype(jnp.float32)
    sx1b = _rowb(sx1, RT)
    sy1b = _rowb(sy1, RT)
    sx2b = _rowb(sx2, RT)
    sy2b = _rowb(sy2, RT)
    areab = _rowb(area_r, RT)
    jio2 = jax.lax.broadcasted_iota(jnp.int32, (RT, NSEL), 1).astype(jnp.float32)
    for rt in range(NSEL // RT):
        sl = slice(rt * RT, (rt + 1) * RT)
        x1c = jnp.transpose(sx1[:, sl])
        y1c = jnp.transpose(sy1[:, sl])
        x2c = jnp.transpose(sx2[:, sl])
        y2c = jnp.transpose(sy2[:, sl])
        ac = jnp.transpose(area_r[:, sl])
        ltx = jnp.maximum(x1c, sx1b)
        lty = jnp.maximum(y1c, sy1b)
        rbx = jnp.minimum(x2c, sx2b)
        rby = jnp.minimum(y2c, sy2b)
        wh_x = jnp.maximum(rbx - ltx, 0.0)
        wh_y = jnp.maximum(rby - lty, 0.0)
        inter = wh_x * wh_y
        iou = inter / (ac + areab - inter + 1e-9)
        iio = jax.lax.broadcasted_iota(jnp.int32, (RT, 1), 0).astype(jnp.float32) + float(rt * RT)
        supt = ((iou > NMS_THRESH) & (iio < jio2)).astype(jnp.bfloat16)
        sup_ref[pl.ds(rt * RT, RT), :] = supt

    valid = (jio < float(PRE_NMS_TOPK))
    keep0 = valid.astype(jnp.bfloat16)

    def nms_cond(carry):
        _, changed, it = carry
        return changed & (it < NSEL)

    def nms_body(carry):
        keep, _, it = carry
        supd = jax.lax.dot_general(keep, sup_ref[...], (((1,), (0,)), ((), ())),
                                   preferred_element_type=jnp.float32)
        new = (valid & (supd == 0.0)).astype(jnp.bfloat16)
        delta = jnp.sum(jnp.abs(new.astype(jnp.float32) - keep.astype(jnp.float32)))
        return new, delta > 0.0, it + 1

    keep, _, _ = jax.lax.while_loop(
        nms_cond, nms_body, (keep0, jnp.bool_(True), jnp.int32(0)))
    keepf = keep.astype(jnp.float32)
    validf = valid.astype(jnp.float32)
    suppf = validf - keepf
    n_keep = jnp.sum(keepf)
    kept_pre = _excl_cumsum_lanes(keepf)
    supp_pre = _excl_cumsum_lanes(suppf)
    posf = (keepf * kept_pre + suppf * (n_keep + supp_pre)
            + (1.0 - validf) * 3000.0)

    FT = 256
    posfb = _rowb(posf, FT)
    fparts = []
    for ft in range(NOUT // FT):
        f0 = float(ft * FT)
        fio = jax.lax.broadcasted_iota(jnp.int32, (FT, 1), 0).astype(jnp.float32) + f0
        oh = (fio == posfb).astype(jnp.float32)
        fparts.append(jax.lax.dot_general(
            oh, sortedT, (((1,), (0,)), ((), ())),
            preferred_element_type=jnp.float32,
            precision=jax.lax.Precision.HIGHEST))
    fin = jnp.concatenate(fparts, axis=0)
    slot = jax.lax.broadcasted_iota(jnp.int32, (NOUT, 1), 0).astype(jnp.float32)
    fsc = jnp.where(slot < n_keep, fin[:, 4:5], -jnp.inf)
    out_ref[...] = jnp.concatenate(
        [fin[:, 0:4], fsc, jnp.zeros((NOUT, 3), jnp.float32)], axis=1)


def kernel(feat, image_shapes, conv_w, conv_b, cls_w, cls_b, reg_w, reg_b):
    B, C, H, W = feat.shape
    P = H * W
    PP = 2560
    NA = P * NUM_ANCHORS
    NTOT = 22528
    n9 = NUM_ANCHORS

    xpad = jnp.pad(feat[0], ((0, 0), (1, 1), (1, 1)))
    taps = [xpad[:, ky:ky + H, kx:kx + W].reshape(C, P)
            for ky in range(3) for kx in range(3)]
    xs = jnp.pad(jnp.stack(taps, 0), ((0, 0), (0, 0), (0, PP - P)))
    xcolT = jnp.transpose(xs.reshape(9 * C, PP))
    wcol = jnp.transpose(conv_w, (2, 3, 1, 0)).reshape(9 * C, C)

    rpnT = pl.pallas_call(
        _trunk_body,
        out_shape=jax.ShapeDtypeStruct((PP, C), jnp.float32),
    )(xcolT, wcol, conv_b.reshape(1, C))

    cls9, reg36 = pl.pallas_call(
        _heads_body,
        out_shape=[jax.ShapeDtypeStruct((n9, PP), jnp.float32),
                   jax.ShapeDtypeStruct((4 * n9, PP), jnp.float32)],
    )(rpnT, cls_w.reshape(n9, C), cls_b.reshape(n9, 1),
      reg_w.reshape(4 * n9, C), reg_b.reshape(4 * n9, 1))

    cls_flat = jnp.transpose(cls9[:, :P]).reshape(1, NA)
    reg_flat = jnp.transpose(
        reg36[:, :P].reshape(n9, 4, P), (1, 2, 0)).reshape(4, NA)
    anc = jnp.asarray(np.pad(
        np.transpose(_np_anchors_out(H, W)), ((0, 0), (0, NTOT - NA))))
    imgf = image_shapes.astype(jnp.float32)

    fin = pl.pallas_call(
        _select_body,
        out_shape=jax.ShapeDtypeStruct((NOUT, 8), jnp.float32),
        scratch_shapes=[pltpu.VMEM((NSEL, NSEL), jnp.bfloat16)],
    )(cls_flat, reg_flat, anc, imgf)

    final_boxes = fin[:POST_NMS_TOPK, 0:4][None]
    final_scores = fin[:POST_NMS_TOPK, 4][None]
    cls_logits = cls_flat.reshape(B, NA)
    reg_deltas = jnp.transpose(reg36[:, :P]).reshape(B, NA, 4)
    anchors = jnp.asarray(_np_anchors_out(H, W))
    return (final_boxes, final_scores, cls_logits, reg_deltas, anchors)
